# baseline (speedup 1.0000x reference)
import functools

import jax
import jax.numpy as jnp
from jax.experimental import pallas as pl

N = 100000
D = 128
G = 512
BLK = 2048


def _mlp_segsum_kernel(x_ref, idx_ref, w0_ref, w1_ref, w2_ref, w3_ref, w4_ref,
                       b0_ref, b1_ref, b2_ref, b3_ref, b4_ref, out_ref):
    h = x_ref[...].astype(jnp.bfloat16)
    for w_ref, b_ref in ((w0_ref, b0_ref), (w1_ref, b1_ref),
                         (w2_ref, b2_ref), (w3_ref, b3_ref)):
        h = jnp.dot(h, w_ref[...].astype(jnp.bfloat16),
                    preferred_element_type=jnp.float32)
        h = h + b_ref[...]
        h = (h * jax.nn.sigmoid(h)).astype(jnp.bfloat16)
    s = jnp.dot(h, w4_ref[...].astype(jnp.bfloat16),
                preferred_element_type=jnp.float32)
    s = s + b4_ref[...]

    idx = idx_ref[...]
    gids = jax.lax.broadcasted_iota(jnp.int32, (BLK, G), 1)
    masked = jnp.where(idx == gids, s, 0.0)
    contrib = jnp.sum(masked, axis=0, keepdims=True)

    @pl.when(pl.program_id(0) == 0)
    def _():
        out_ref[...] = jnp.zeros_like(out_ref)

    out_ref[...] += contrib


@jax.jit
def kernel(node_features, batch_idx, W0, W1, W2, W3, W4, b0, b1, b2, b3, b4):
    n_blocks = pl.cdiv(N, BLK)
    n_pad = n_blocks * BLK - N
    x = jnp.pad(node_features, ((0, n_pad), (0, 0)))
    idx = jnp.pad(batch_idx.astype(jnp.int32), (0, n_pad),
                  constant_values=-1).reshape(-1, 1)

    out = pl.pallas_call(
        _mlp_segsum_kernel,
        grid=(n_blocks,),
        in_specs=[
            pl.BlockSpec((BLK, D), lambda i: (i, 0)),
            pl.BlockSpec((BLK, 1), lambda i: (i, 0)),
            pl.BlockSpec((D, D), lambda i: (0, 0)),
            pl.BlockSpec((D, D), lambda i: (0, 0)),
            pl.BlockSpec((D, D), lambda i: (0, 0)),
            pl.BlockSpec((D, D), lambda i: (0, 0)),
            pl.BlockSpec((D, 1), lambda i: (0, 0)),
            pl.BlockSpec((1, D), lambda i: (0, 0)),
            pl.BlockSpec((1, D), lambda i: (0, 0)),
            pl.BlockSpec((1, D), lambda i: (0, 0)),
            pl.BlockSpec((1, D), lambda i: (0, 0)),
            pl.BlockSpec((1, 1), lambda i: (0, 0)),
        ],
        out_specs=pl.BlockSpec((1, G), lambda i: (0, 0)),
        out_shape=jax.ShapeDtypeStruct((1, G), jnp.float32),
    )(x, idx, W0, W1, W2, W3, W4,
      b0.reshape(1, D), b1.reshape(1, D), b2.reshape(1, D), b3.reshape(1, D),
      b4.reshape(1, 1))
    return out.reshape(G)


# speedup vs baseline: 1.0742x; 1.0742x over previous
import functools

import jax
import jax.numpy as jnp
from jax.experimental import pallas as pl

N = 100000
D = 128
G = 512
BLK = 2048


def _mlp_segsum_kernel(x_ref, idx_ref, w0_ref, w1_ref, w2_ref, w3_ref, w4_ref,
                       b0_ref, b1_ref, b2_ref, b3_ref, b4_ref, out_ref):
    h = x_ref[...].astype(jnp.bfloat16)
    for w_ref, b_ref in ((w0_ref, b0_ref), (w1_ref, b1_ref),
                         (w2_ref, b2_ref), (w3_ref, b3_ref)):
        h = jnp.dot(h, w_ref[...].astype(jnp.bfloat16),
                    preferred_element_type=jnp.float32)
        t = (h + b_ref[...]) * 0.5
        h = (t * (1.0 + jnp.tanh(t))).astype(jnp.bfloat16)
    s = jnp.dot(h, w4_ref[...].astype(jnp.bfloat16),
                preferred_element_type=jnp.float32)
    s = s + b4_ref[...]

    idx = idx_ref[...]
    gids = jax.lax.broadcasted_iota(jnp.int32, (BLK, G), 1)
    masked = jnp.where(idx == gids, s, 0.0)
    contrib = jnp.sum(masked, axis=0, keepdims=True)

    @pl.when(pl.program_id(0) == 0)
    def _():
        out_ref[...] = jnp.zeros_like(out_ref)

    out_ref[...] += contrib


@jax.jit
def kernel(node_features, batch_idx, W0, W1, W2, W3, W4, b0, b1, b2, b3, b4):
    n_blocks = pl.cdiv(N, BLK)
    n_pad = n_blocks * BLK - N
    x = jnp.pad(node_features, ((0, n_pad), (0, 0)))
    idx = jnp.pad(batch_idx.astype(jnp.int32), (0, n_pad),
                  constant_values=-1).reshape(-1, 1)

    out = pl.pallas_call(
        _mlp_segsum_kernel,
        grid=(n_blocks,),
        in_specs=[
            pl.BlockSpec((BLK, D), lambda i: (i, 0)),
            pl.BlockSpec((BLK, 1), lambda i: (i, 0)),
            pl.BlockSpec((D, D), lambda i: (0, 0)),
            pl.BlockSpec((D, D), lambda i: (0, 0)),
            pl.BlockSpec((D, D), lambda i: (0, 0)),
            pl.BlockSpec((D, D), lambda i: (0, 0)),
            pl.BlockSpec((D, 1), lambda i: (0, 0)),
            pl.BlockSpec((1, D), lambda i: (0, 0)),
            pl.BlockSpec((1, D), lambda i: (0, 0)),
            pl.BlockSpec((1, D), lambda i: (0, 0)),
            pl.BlockSpec((1, D), lambda i: (0, 0)),
            pl.BlockSpec((1, 1), lambda i: (0, 0)),
        ],
        out_specs=pl.BlockSpec((1, G), lambda i: (0, 0)),
        out_shape=jax.ShapeDtypeStruct((1, G), jnp.float32),
    )(x, idx, W0, W1, W2, W3, W4,
      b0.reshape(1, D), b1.reshape(1, D), b2.reshape(1, D), b3.reshape(1, D),
      b4.reshape(1, 1))
    return out.reshape(G)


# speedup vs baseline: 1.1027x; 1.0266x over previous
import functools

import jax
import jax.numpy as jnp
from jax.experimental import pallas as pl

N = 100000
D = 128
G = 512
BLK = 2000


def _mlp_segsum_kernel(x_ref, idx_ref, w0_ref, w1_ref, w2_ref, w3_ref, w4_ref,
                       b0_ref, b1_ref, b2_ref, b3_ref, b4_ref, out_ref):
    h = x_ref[...].astype(jnp.bfloat16)
    for w_ref, b_ref in ((w0_ref, b0_ref), (w1_ref, b1_ref),
                         (w2_ref, b2_ref), (w3_ref, b3_ref)):
        h = jnp.dot(h, w_ref[...].astype(jnp.bfloat16),
                    preferred_element_type=jnp.float32)
        t = (h + b_ref[...]) * 0.5
        h = (t * (1.0 + jnp.tanh(t))).astype(jnp.bfloat16)
    s = jnp.dot(h, w4_ref[...].astype(jnp.bfloat16),
                preferred_element_type=jnp.float32)
    s = s + b4_ref[...]

    idx = idx_ref[...]
    gids = jax.lax.broadcasted_iota(jnp.int32, (BLK, G), 1)
    masked = jnp.where(idx == gids, s, 0.0)
    contrib = jnp.sum(masked, axis=0, keepdims=True)

    @pl.when(pl.program_id(0) == 0)
    def _():
        out_ref[...] = jnp.zeros_like(out_ref)

    out_ref[...] += contrib


@jax.jit
def kernel(node_features, batch_idx, W0, W1, W2, W3, W4, b0, b1, b2, b3, b4):
    n_blocks = N // BLK
    x = node_features
    idx = batch_idx.astype(jnp.int32).reshape(-1, 1)

    out = pl.pallas_call(
        _mlp_segsum_kernel,
        grid=(n_blocks,),
        in_specs=[
            pl.BlockSpec((BLK, D), lambda i: (i, 0)),
            pl.BlockSpec((BLK, 1), lambda i: (i, 0)),
            pl.BlockSpec((D, D), lambda i: (0, 0)),
            pl.BlockSpec((D, D), lambda i: (0, 0)),
            pl.BlockSpec((D, D), lambda i: (0, 0)),
            pl.BlockSpec((D, D), lambda i: (0, 0)),
            pl.BlockSpec((D, 1), lambda i: (0, 0)),
            pl.BlockSpec((1, D), lambda i: (0, 0)),
            pl.BlockSpec((1, D), lambda i: (0, 0)),
            pl.BlockSpec((1, D), lambda i: (0, 0)),
            pl.BlockSpec((1, D), lambda i: (0, 0)),
            pl.BlockSpec((1, 1), lambda i: (0, 0)),
        ],
        out_specs=pl.BlockSpec((1, G), lambda i: (0, 0)),
        out_shape=jax.ShapeDtypeStruct((1, G), jnp.float32),
    )(x, idx, W0, W1, W2, W3, W4,
      b0.reshape(1, D), b1.reshape(1, D), b2.reshape(1, D), b3.reshape(1, D),
      b4.reshape(1, 1))
    return out.reshape(G)


# speedup vs baseline: 1.3128x; 1.1905x over previous
import functools

import jax
import jax.numpy as jnp
from jax.experimental import pallas as pl

N = 100000
D = 128
G = 512
BLK = 4000


def _mlp_segsum_kernel(x_ref, idx_ref, w0_ref, w1_ref, w2_ref, w3_ref, w4_ref,
                       b0_ref, b1_ref, b2_ref, b3_ref, b4_ref, out_ref):
    h = x_ref[...].astype(jnp.bfloat16)
    for w_ref, b_ref in ((w0_ref, b0_ref), (w1_ref, b1_ref),
                         (w2_ref, b2_ref), (w3_ref, b3_ref)):
        h = jnp.dot(h, w_ref[...].astype(jnp.bfloat16),
                    preferred_element_type=jnp.float32)
        t = (h + b_ref[...]) * 0.5
        h = (t * (1.0 + jnp.tanh(t))).astype(jnp.bfloat16)
    s = jnp.dot(h, w4_ref[...].astype(jnp.bfloat16),
                preferred_element_type=jnp.float32)
    s = s + b4_ref[...]

    idx = idx_ref[...]
    gids = jax.lax.broadcasted_iota(jnp.int32, (BLK, G), 1)
    masked = jnp.where(idx == gids, s, 0.0)
    contrib = jnp.sum(masked, axis=0, keepdims=True)

    @pl.when(pl.program_id(0) == 0)
    def _():
        out_ref[...] = jnp.zeros_like(out_ref)

    out_ref[...] += contrib


@jax.jit
def kernel(node_features, batch_idx, W0, W1, W2, W3, W4, b0, b1, b2, b3, b4):
    n_blocks = N // BLK
    x = node_features
    idx = batch_idx.astype(jnp.int32).reshape(-1, 1)

    out = pl.pallas_call(
        _mlp_segsum_kernel,
        grid=(n_blocks,),
        in_specs=[
            pl.BlockSpec((BLK, D), lambda i: (i, 0)),
            pl.BlockSpec((BLK, 1), lambda i: (i, 0)),
            pl.BlockSpec((D, D), lambda i: (0, 0)),
            pl.BlockSpec((D, D), lambda i: (0, 0)),
            pl.BlockSpec((D, D), lambda i: (0, 0)),
            pl.BlockSpec((D, D), lambda i: (0, 0)),
            pl.BlockSpec((D, 1), lambda i: (0, 0)),
            pl.BlockSpec((1, D), lambda i: (0, 0)),
            pl.BlockSpec((1, D), lambda i: (0, 0)),
            pl.BlockSpec((1, D), lambda i: (0, 0)),
            pl.BlockSpec((1, D), lambda i: (0, 0)),
            pl.BlockSpec((1, 1), lambda i: (0, 0)),
        ],
        out_specs=pl.BlockSpec((1, G), lambda i: (0, 0)),
        out_shape=jax.ShapeDtypeStruct((1, G), jnp.float32),
    )(x, idx, W0, W1, W2, W3, W4,
      b0.reshape(1, D), b1.reshape(1, D), b2.reshape(1, D), b3.reshape(1, D),
      b4.reshape(1, 1))
    return out.reshape(G)
